# baseline (speedup 1.0000x reference)
import functools

import jax
import jax.numpy as jnp
from jax import lax
from jax.experimental import pallas as pl
from jax.experimental.pallas import tpu as pltpu
from jax.experimental.pallas import tpu_sc as plsc

_NC = 2
_NS = 16
_NW = _NC * _NS
_L = 16
_CHUNK = 128


def _wmf_body(user_hbm, item_hbm, weight_hbm, ut_hbm, it_hbm, out_hbm,
              idx_u, idx_i, rows_u, rows_i, w_v, out_v, sem):
    nchunks = idx_u.shape[0]
    bw = nchunks * _CHUNK
    nf = ut_hbm.shape[1]
    wid = lax.axis_index("s") * _NC + lax.axis_index("c")
    base = wid * bw

    pltpu.sync_copy(user_hbm.at[pl.ds(wid * nchunks, nchunks)], idx_u)
    pltpu.sync_copy(item_hbm.at[pl.ds(wid * nchunks, nchunks)], idx_i)
    pltpu.sync_copy(weight_hbm.at[pl.ds(base, bw)], w_v)

    copies = []
    for j in range(nchunks):
        copies.append(pltpu.async_copy(
            ut_hbm.at[idx_u.at[j]], rows_u.at[pl.ds(j * _CHUNK, _CHUNK)], sem))
        copies.append(pltpu.async_copy(
            it_hbm.at[idx_i.at[j]], rows_i.at[pl.ds(j * _CHUNK, _CHUNK)], sem))
    for c in copies:
        c.wait()

    lane = lax.iota(jnp.int32, _L)

    def body(g, carry):
        rvec = lane + g * _L
        acc = jnp.zeros((_L,), jnp.float32)
        for f in range(nf):
            fvec = jnp.full((_L,), f, jnp.int32)
            u = plsc.load_gather(rows_u, [rvec, fvec])
            v = plsc.load_gather(rows_i, [rvec, fvec])
            acc = acc + u * v
        x = acc * w_v[pl.ds(g * _L, _L)]
        out_v[pl.ds(g * _L, _L)] = 1.0 / (1.0 + jnp.exp(-x))
        return carry

    lax.fori_loop(0, bw // _L, body, 0)
    pltpu.sync_copy(out_v, out_hbm.at[pl.ds(base, bw)])


@jax.jit
def kernel(user, item, weight, user_table, item_table):
    b = user.shape[0]
    nf = user_table.shape[1]
    bw = b // _NW
    nchunks = bw // _CHUNK
    user2 = user.reshape(_NW * nchunks, _CHUNK).astype(jnp.int32)
    item2 = item.reshape(_NW * nchunks, _CHUNK).astype(jnp.int32)

    mesh = plsc.VectorSubcoreMesh(core_axis_name="c", subcore_axis_name="s")
    run = pl.kernel(
        _wmf_body,
        out_type=jax.ShapeDtypeStruct((b,), jnp.float32),
        mesh=mesh,
        compiler_params=pltpu.CompilerParams(
            needs_layout_passes=False, use_tc_tiling_on_sc=False),
        scratch_types=[
            pltpu.VMEM((nchunks, _CHUNK), jnp.int32),
            pltpu.VMEM((nchunks, _CHUNK), jnp.int32),
            pltpu.VMEM((bw, nf), jnp.float32),
            pltpu.VMEM((bw, nf), jnp.float32),
            pltpu.VMEM((bw,), jnp.float32),
            pltpu.VMEM((bw,), jnp.float32),
            pltpu.SemaphoreType.DMA,
        ],
    )
    return run(user2, item2, weight, user_table, item_table)


# speedup vs baseline: 3.9989x; 3.9989x over previous
import jax
import jax.numpy as jnp
from jax import lax
from jax.experimental import pallas as pl
from jax.experimental.pallas import tpu as pltpu
from jax.experimental.pallas import tpu_sc as plsc

_NC = 2
_NS = 16
_NW = _NC * _NS
_L = 16
_NBUF = 8
_LAST_FULL = 7811
_TAIL = (_LAST_FULL + 1) * 128
_N = 1000000
_TAILBASE = _N - 128
_BW = 512


def _fire(t_hbm, blk, sem, slot, col):
    c = jnp.minimum(col >> 7, _LAST_FULL)
    base = pl.multiple_of(c * 128, 128)
    pltpu.async_copy(t_hbm.at[:, pl.ds(base, 128)], blk.at[slot], sem.at[slot])


def _drain(t_hbm, blk, sem, slot):
    pltpu.make_async_copy(t_hbm.at[:, pl.ds(0, 128)], blk.at[slot],
                          sem.at[slot]).wait()


def _wmf_body(user_hbm, item_hbm, weight_hbm, ut_hbm, it_hbm, tu_hbm, ti_hbm,
              out_hbm, idx_us, idx_is, w_v, out_v, blku, blki, sem_u, sem_i):
    wid = lax.axis_index("s") * _NC + lax.axis_index("c")
    base = wid * _BW

    pltpu.sync_copy(user_hbm.at[pl.ds(base, _BW)], idx_us)
    pltpu.sync_copy(item_hbm.at[pl.ds(base, _BW)], idx_is)
    pltpu.sync_copy(weight_hbm.at[pl.ds(base, _BW)], w_v)
    pltpu.sync_copy(tu_hbm, blku.at[_NBUF])
    pltpu.sync_copy(ti_hbm, blki.at[_NBUF])

    fvec = lax.iota(jnp.int32, _L)
    f2 = fvec + _L

    iu0 = idx_us[pl.ds(0, _L)]
    ii0 = idx_is[pl.ds(0, _L)]
    for k in range(_NBUF):
        _fire(ut_hbm, blku, sem_u, k, iu0[k])
        _fire(it_hbm, blki, sem_i, k, ii0[k])

    def _dot_one(cu, ci, slot):
        tail_u = cu >= _TAIL
        tail_i = ci >= _TAIL
        slot_u = jnp.full((_L,), jnp.where(tail_u, _NBUF, slot), jnp.int32)
        slot_i = jnp.full((_L,), jnp.where(tail_i, _NBUF, slot), jnp.int32)
        su = jnp.full((_L,), jnp.where(tail_u, cu - _TAILBASE, cu & 127),
                      jnp.int32)
        si = jnp.full((_L,), jnp.where(tail_i, ci - _TAILBASE, ci & 127),
                      jnp.int32)
        u0 = plsc.load_gather(blku, [slot_u, fvec, su])
        u1 = plsc.load_gather(blku, [slot_u, f2, su])
        v0 = plsc.load_gather(blki, [slot_i, fvec, si])
        v1 = plsc.load_gather(blki, [slot_i, f2, si])
        return lax.reduce_sum(u0 * v0 + u1 * v1, axes=(0,))

    def body(g, carry):
        gb = pl.multiple_of(g * _L, _L)
        iu = idx_us[pl.ds(gb, _L)]
        ii = idx_is[pl.ds(gb, _L)]
        gn = jnp.minimum(g + 1, _BW // _L - 1)
        gnb = pl.multiple_of(gn * _L, _L)
        iun = idx_us[pl.ds(gnb, _L)]
        iin = idx_is[pl.ds(gnb, _L)]

        acc = jnp.zeros((_L,), jnp.float32)
        for k in range(_L):
            slot = k & (_NBUF - 1)
            _drain(ut_hbm, blku, sem_u, slot)
            _drain(it_hbm, blki, sem_i, slot)
            dot = _dot_one(iu[k], ii[k], slot)
            acc = jnp.where(fvec == k, jnp.full((_L,), dot, jnp.float32), acc)

            if k < _NBUF:
                _fire(ut_hbm, blku, sem_u, slot, iu[k + _NBUF])
                _fire(it_hbm, blki, sem_i, slot, ii[k + _NBUF])
            else:
                @pl.when(g < _BW // _L - 1)
                def _():
                    _fire(ut_hbm, blku, sem_u, slot, iun[k - _NBUF])
                    _fire(it_hbm, blki, sem_i, slot, iin[k - _NBUF])

        x = acc * w_v[pl.ds(gb, _L)]
        out_v[pl.ds(gb, _L)] = 1.0 / (1.0 + jnp.exp(-x))
        return carry

    lax.fori_loop(0, _BW // _L, body, 0)
    pltpu.sync_copy(out_v, out_hbm.at[pl.ds(base, _BW)])


@jax.jit
def kernel(user, item, weight, user_table, item_table):
    b = user.shape[0]
    ut = user_table.T
    it = item_table.T
    tail_u = ut[:, _TAILBASE:]
    tail_i = it[:, _TAILBASE:]

    mesh = plsc.VectorSubcoreMesh(core_axis_name="c", subcore_axis_name="s")
    run = pl.kernel(
        _wmf_body,
        out_type=jax.ShapeDtypeStruct((b,), jnp.float32),
        mesh=mesh,
        compiler_params=pltpu.CompilerParams(needs_layout_passes=False),
        scratch_types=[
            pltpu.VMEM((_BW,), jnp.int32),
            pltpu.VMEM((_BW,), jnp.int32),
            pltpu.VMEM((_BW,), jnp.float32),
            pltpu.VMEM((_BW,), jnp.float32),
            pltpu.VMEM((_NBUF + 1, 32, 128), jnp.float32),
            pltpu.VMEM((_NBUF + 1, 32, 128), jnp.float32),
            pltpu.SemaphoreType.DMA((_NBUF,)),
            pltpu.SemaphoreType.DMA((_NBUF,)),
        ],
    )
    return run(user.astype(jnp.int32), item.astype(jnp.int32), weight,
               ut, it, tail_u, tail_i)
